# baseline (speedup 1.0000x reference)
import functools

import jax
import jax.numpy as jnp
from jax import lax
from jax.experimental import pallas as pl
from jax.experimental.pallas import tpu as pltpu
from jax.experimental.pallas import tpu_sc as plsc

R, C = 128, 32768
NC, NS, L = 2, 16, 16
NW = NC * NS
R_SC = 64
R_TC = R - R_SC
RPW = R_SC // NW
K = 16
NVC = C // L

_mesh = plsc.VectorSubcoreMesh(
    core_axis_name="c", subcore_axis_name="s", num_cores=NC
)


def _merge(a, b):
    va, ca = a
    vb, cb = b
    take = (vb > va) | ((vb == va) & (cb < ca))
    return jnp.where(take, vb, va), jnp.where(take, cb, ca)


@functools.partial(
    pl.kernel,
    out_type=jax.ShapeDtypeStruct((NW // 4, 4, L), jnp.int32),
    mesh=_mesh,
    scratch_types=[
        pltpu.VMEM((C,), jnp.float32),
        pltpu.VMEM((C,), jnp.float32),
        pltpu.VMEM((L,), jnp.int32),
        pltpu.SemaphoreType.DMA,
        pltpu.SemaphoreType.DMA,
    ],
)
def _argmax_sc(x_hbm, out_hbm, buf_a, buf_b, res_v, sem_a, sem_b):
    cid = lax.axis_index("c")
    sid = lax.axis_index("s")
    wid = cid * NS + sid
    row0 = R_TC + wid * RPW
    bufs = (buf_a, buf_b)
    sems = (sem_a, sem_b)
    lane = jnp.arange(L, dtype=jnp.int32)

    copies = [
        pltpu.async_copy(x_hbm.at[row0 + r], bufs[r], sems[r])
        for r in range(RPW)
    ]

    resvec = jnp.zeros((L,), jnp.int32)
    for r in range(RPW):
        buf = bufs[r % 2]
        copies[r % 2].wait()

        init = tuple(
            (jnp.full((L,), -jnp.inf, jnp.float32), jnp.zeros((L,), jnp.int32))
            for _ in range(K)
        )

        def body(i, accs, buf=buf):
            base = i * (K * L)
            ib = jnp.full((L,), i, jnp.int32)
            out = []
            for u in range(K):
                vmax, vidx = accs[u]
                xv = buf[pl.ds(base + u * L, L)]
                gt = xv > vmax
                out.append((jnp.where(gt, xv, vmax), jnp.where(gt, ib, vidx)))
            return tuple(out)

        accs = plsc.parallel_loop(0, NVC // K, carry=init, unroll=4)(body)

        pairs = [
            (vmax, vidx * (K * L) + u * L + lane)
            for u, (vmax, vidx) in enumerate(accs)
        ]
        while len(pairs) > 1:
            pairs = [_merge(pairs[j], pairs[j + 1]) for j in range(0, len(pairs), 2)]

        vals, idxs = pairs[0]
        for sh in (8, 4, 2, 1):
            perm = lane ^ sh
            ov = vals.at[perm].get(mode="promise_in_bounds")
            oi = idxs.at[perm].get(mode="promise_in_bounds")
            vals, idxs = _merge((vals, idxs), (ov, oi))
        resvec = jnp.where(lane == RPW * (wid % 4) + r, idxs, resvec)

    res_v[...] = resvec
    pltpu.sync_copy(res_v, out_hbm.at[wid // 4, wid % 4])


RB = 8
NG = 16
GW = NG * 128
NOC = C // GW
NRB = R_TC // RB


def _tc_body(x_ref, o_ref):
    lane = lax.broadcasted_iota(jnp.int32, (RB, 128), 1)
    accs = [
        (jnp.full((RB, 128), -jnp.inf, jnp.float32), jnp.zeros((RB, 128), jnp.int32))
        for _ in range(NG)
    ]
    for c in range(NOC):
        for k in range(NG):
            xv = x_ref[:, c * GW + k * 128:c * GW + (k + 1) * 128]
            vmax, vc = accs[k]
            gt = xv > vmax
            accs[k] = (jnp.where(gt, xv, vmax), jnp.where(gt, c, vc))

    pairs = [(v, vc * GW + k * 128 + lane) for k, (v, vc) in enumerate(accs)]
    while len(pairs) > 1:
        pairs = [_merge(pairs[j], pairs[j + 1]) for j in range(0, len(pairs), 2)]
    vals, cols = pairs[0]
    rowmax = jnp.max(vals, axis=1, keepdims=True)
    cand = jnp.where(vals == rowmax, cols, jnp.int32(2**31 - 1))
    argcol = jnp.min(cand, axis=1, keepdims=True)
    o_ref[...] = argcol.reshape(1, 1, RB)


_argmax_tc = pl.pallas_call(
    _tc_body,
    grid=(NRB,),
    in_specs=[pl.BlockSpec((RB, C), lambda i: (i, 0))],
    out_specs=pl.BlockSpec((1, 1, RB), lambda i: (i, 0, 0)),
    out_shape=jax.ShapeDtypeStruct((NRB, 1, RB), jnp.int32),
    compiler_params=pltpu.CompilerParams(
        dimension_semantics=("parallel",),
    ),
)


def _asm_body(tc_ref, sc_ref, o_ref):
    tci = tc_ref[...][:, 0, :]
    s = sc_ref[...]
    sci = (s[:, 0, :RB] + s[:, 1, :RB] + s[:, 2, :RB] + s[:, 3, :RB])
    o_ref[pl.ds(0, NRB), :] = tci
    o_ref[pl.ds(NRB, NRB), :] = sci


_assemble = pl.pallas_call(
    _asm_body,
    out_shape=jax.ShapeDtypeStruct((2 * NRB, RB), jnp.int32),
)


def kernel(x):
    sc_out = _argmax_sc(x)
    tc_out = _argmax_tc(x)
    return _assemble(tc_out, sc_out).reshape(R)


# speedup vs baseline: 1.0017x; 1.0017x over previous
import functools

import jax
import jax.numpy as jnp
from jax import lax
from jax.experimental import pallas as pl
from jax.experimental.pallas import tpu as pltpu
from jax.experimental.pallas import tpu_sc as plsc

R, C = 128, 32768
NC, NS, L = 2, 16, 16
NW = NC * NS
R_SC = 64
R_TC = R - R_SC
RPW = R_SC // NW
K = 16
NVC = C // L

_mesh = plsc.VectorSubcoreMesh(
    core_axis_name="c", subcore_axis_name="s", num_cores=NC
)


def _merge(a, b):
    va, ca = a
    vb, cb = b
    take = (vb > va) | ((vb == va) & (cb < ca))
    return jnp.where(take, vb, va), jnp.where(take, cb, ca)


@functools.partial(
    pl.kernel,
    out_type=jax.ShapeDtypeStruct((NW // 4, 4, L), jnp.int32),
    mesh=_mesh,
    scratch_types=[
        pltpu.VMEM((C,), jnp.float32),
        pltpu.VMEM((C,), jnp.float32),
        pltpu.VMEM((L,), jnp.int32),
        pltpu.SemaphoreType.DMA,
        pltpu.SemaphoreType.DMA,
    ],
)
def _argmax_sc(x_hbm, out_hbm, buf_a, buf_b, res_v, sem_a, sem_b):
    cid = lax.axis_index("c")
    sid = lax.axis_index("s")
    wid = cid * NS + sid
    row0 = R_TC + wid * RPW
    bufs = (buf_a, buf_b)
    sems = (sem_a, sem_b)
    lane = jnp.arange(L, dtype=jnp.int32)

    copies = [
        pltpu.async_copy(x_hbm.at[row0 + r], bufs[r], sems[r])
        for r in range(RPW)
    ]

    resvec = jnp.zeros((L,), jnp.int32)
    for r in range(RPW):
        buf = bufs[r % 2]
        copies[r % 2].wait()

        init = tuple(
            (jnp.full((L,), -jnp.inf, jnp.float32), jnp.zeros((L,), jnp.int32))
            for _ in range(K)
        )

        def body(i, accs, buf=buf):
            base = i * (K * L)
            ib = jnp.full((L,), i, jnp.int32)
            out = []
            for u in range(K):
                vmax, vidx = accs[u]
                xv = buf[pl.ds(base + u * L, L)]
                gt = xv > vmax
                out.append((jnp.where(gt, xv, vmax), jnp.where(gt, ib, vidx)))
            return tuple(out)

        accs = plsc.parallel_loop(0, NVC // K, carry=init, unroll=2)(body)

        pairs = [
            (vmax, vidx * (K * L) + u * L + lane)
            for u, (vmax, vidx) in enumerate(accs)
        ]
        while len(pairs) > 1:
            pairs = [_merge(pairs[j], pairs[j + 1]) for j in range(0, len(pairs), 2)]

        vals, idxs = pairs[0]
        for sh in (8, 4, 2, 1):
            perm = lane ^ sh
            ov = vals.at[perm].get(mode="promise_in_bounds")
            oi = idxs.at[perm].get(mode="promise_in_bounds")
            vals, idxs = _merge((vals, idxs), (ov, oi))
        resvec = jnp.where(lane == RPW * (wid % 4) + r, idxs, resvec)

    res_v[...] = resvec
    pltpu.sync_copy(res_v, out_hbm.at[wid // 4, wid % 4])


RB = 8
NG = 16
GW = NG * 128
NOC = C // GW
NRB = R_TC // RB


def _tc_body(x_ref, o_ref):
    lane = lax.broadcasted_iota(jnp.int32, (RB, 128), 1)
    accs = [
        (jnp.full((RB, 128), -jnp.inf, jnp.float32), jnp.zeros((RB, 128), jnp.int32))
        for _ in range(NG)
    ]
    for c in range(NOC):
        for k in range(NG):
            xv = x_ref[:, c * GW + k * 128:c * GW + (k + 1) * 128]
            vmax, vc = accs[k]
            gt = xv > vmax
            accs[k] = (jnp.where(gt, xv, vmax), jnp.where(gt, c, vc))

    pairs = [(v, vc * GW + k * 128 + lane) for k, (v, vc) in enumerate(accs)]
    while len(pairs) > 1:
        pairs = [_merge(pairs[j], pairs[j + 1]) for j in range(0, len(pairs), 2)]
    vals, cols = pairs[0]
    rowmax = jnp.max(vals, axis=1, keepdims=True)
    cand = jnp.where(vals == rowmax, cols, jnp.int32(2**31 - 1))
    argcol = jnp.min(cand, axis=1, keepdims=True)
    o_ref[...] = argcol.reshape(1, 1, RB)


_argmax_tc = pl.pallas_call(
    _tc_body,
    grid=(NRB,),
    in_specs=[pl.BlockSpec((RB, C), lambda i: (i, 0))],
    out_specs=pl.BlockSpec((1, 1, RB), lambda i: (i, 0, 0)),
    out_shape=jax.ShapeDtypeStruct((NRB, 1, RB), jnp.int32),
    compiler_params=pltpu.CompilerParams(
        dimension_semantics=("parallel",),
    ),
)


def _asm_body(tc_ref, sc_ref, o_ref):
    tci = tc_ref[...][:, 0, :]
    s = sc_ref[...]
    sci = (s[:, 0, :RB] + s[:, 1, :RB] + s[:, 2, :RB] + s[:, 3, :RB])
    o_ref[pl.ds(0, NRB), :] = tci
    o_ref[pl.ds(NRB, NRB), :] = sci


_assemble = pl.pallas_call(
    _asm_body,
    out_shape=jax.ShapeDtypeStruct((2 * NRB, RB), jnp.int32),
)


def kernel(x):
    sc_out = _argmax_sc(x)
    tc_out = _argmax_tc(x)
    return _assemble(tc_out, sc_out).reshape(R)


# speedup vs baseline: 1.0643x; 1.0625x over previous
import functools

import jax
import jax.numpy as jnp
from jax import lax
from jax.experimental import pallas as pl
from jax.experimental.pallas import tpu as pltpu
from jax.experimental.pallas import tpu_sc as plsc

R, C = 128, 32768
NC, NS, L = 2, 16, 16
NW = NC * NS
R_SC = 64
R_TC = R - R_SC
RPW = R_SC // NW
K = 16
NVC = C // L

_mesh = plsc.VectorSubcoreMesh(
    core_axis_name="c", subcore_axis_name="s", num_cores=NC
)


def _merge(a, b):
    va, ca = a
    vb, cb = b
    take = (vb > va) | ((vb == va) & (cb < ca))
    return jnp.where(take, vb, va), jnp.where(take, cb, ca)


@functools.partial(
    pl.kernel,
    out_type=jax.ShapeDtypeStruct((NW, L), jnp.int32),
    mesh=_mesh,
    scratch_types=[
        pltpu.VMEM((C,), jnp.float32),
        pltpu.VMEM((C,), jnp.float32),
        pltpu.VMEM((L,), jnp.int32),
        pltpu.SemaphoreType.DMA,
        pltpu.SemaphoreType.DMA,
    ],
)
def _argmax_sc(x_hbm, out_hbm, buf_a, buf_b, res_v, sem_a, sem_b):
    wid = lax.axis_index("s") * NC + lax.axis_index("c")
    row0 = R_TC + wid * RPW
    bufs = (buf_a, buf_b)
    sems = (sem_a, sem_b)
    lane = jnp.arange(L, dtype=jnp.int32)

    copies = [None, None]
    copies[0] = pltpu.async_copy(x_hbm.at[row0], buf_a, sem_a)

    resvec = jnp.zeros((L,), jnp.int32)
    for r in range(RPW):
        buf = bufs[r % 2]
        copies[r % 2].wait()
        if r + 1 < RPW:
            copies[(r + 1) % 2] = pltpu.async_copy(
                x_hbm.at[row0 + r + 1], bufs[(r + 1) % 2], sems[(r + 1) % 2]
            )

        init = tuple(
            (jnp.full((L,), -jnp.inf, jnp.float32), jnp.zeros((L,), jnp.int32))
            for _ in range(K)
        )

        def body(i, accs, buf=buf):
            base = i * (K * L)
            ib = jnp.full((L,), i, jnp.int32)
            out = []
            for u in range(K):
                vmax, vidx = accs[u]
                xv = buf[pl.ds(base + u * L, L)]
                gt = xv > vmax
                out.append((jnp.where(gt, xv, vmax), jnp.where(gt, ib, vidx)))
            return tuple(out)

        accs = plsc.parallel_loop(0, NVC // K, carry=init, unroll=2)(body)

        pairs = [
            (vmax, vidx * (K * L) + u * L + lane)
            for u, (vmax, vidx) in enumerate(accs)
        ]
        while len(pairs) > 1:
            pairs = [_merge(pairs[j], pairs[j + 1]) for j in range(0, len(pairs), 2)]

        vals, idxs = pairs[0]
        for sh in (8, 4, 2, 1):
            perm = lane ^ sh
            ov = vals.at[perm].get(mode="promise_in_bounds")
            oi = idxs.at[perm].get(mode="promise_in_bounds")
            vals, idxs = _merge((vals, idxs), (ov, oi))
        resvec = jnp.where(lane == r, idxs, resvec)

    res_v[...] = resvec
    pltpu.sync_copy(res_v, out_hbm.at[wid])


RB = 8
NG = 16
GW = NG * 128
NOC = C // GW
NRB = R_TC // RB


def _tc_body(x_ref, o_ref):
    lane = lax.broadcasted_iota(jnp.int32, (RB, 128), 1)
    accs = [
        (jnp.full((RB, 128), -jnp.inf, jnp.float32), jnp.zeros((RB, 128), jnp.int32))
        for _ in range(NG)
    ]
    for c in range(NOC):
        for k in range(NG):
            xv = x_ref[:, c * GW + k * 128:c * GW + (k + 1) * 128]
            vmax, vc = accs[k]
            gt = xv > vmax
            accs[k] = (jnp.where(gt, xv, vmax), jnp.where(gt, c, vc))

    pairs = [(v, vc * GW + k * 128 + lane) for k, (v, vc) in enumerate(accs)]
    while len(pairs) > 1:
        pairs = [_merge(pairs[j], pairs[j + 1]) for j in range(0, len(pairs), 2)]
    vals, cols = pairs[0]
    rowmax = jnp.max(vals, axis=1, keepdims=True)
    cand = jnp.where(vals == rowmax, cols, jnp.int32(2**31 - 1))
    argcol = jnp.min(cand, axis=1, keepdims=True)
    o_ref[...] = jnp.broadcast_to(argcol, (RB, 128))[None]


_argmax_tc = pl.pallas_call(
    _tc_body,
    grid=(NRB,),
    in_specs=[pl.BlockSpec((RB, C), lambda i: (i, 0))],
    out_specs=pl.BlockSpec((1, RB, 128), lambda i: (i, 0, 0)),
    out_shape=jax.ShapeDtypeStruct((NRB, RB, 128), jnp.int32),
    compiler_params=pltpu.CompilerParams(
        dimension_semantics=("parallel",),
    ),
)


def kernel(x):
    sc_out = _argmax_sc(x)
    tc_out = _argmax_tc(x)
    return jnp.concatenate(
        [tc_out[:, :, 0].reshape(R_TC), sc_out[:, :RPW].reshape(R_SC)]
    )
